# Initial kernel scaffold; baseline (speedup 1.0000x reference)
import jax
import jax.numpy as jnp
from jax.experimental import pallas as pl


def kernel(idx, targets, emb):
    raise NotImplementedError("write your pallas kernel here")


# speedup vs baseline: 1.6151x; 1.6151x over previous
import functools

import jax
import jax.numpy as jnp
from jax import lax
from jax.experimental import pallas as pl
from jax.experimental.pallas import tpu as pltpu
from jax.experimental.pallas import tpu_sc as plsc

NC, NS, LANES = 2, 16, 16
NW = NC * NS


def _lse_body(emb_ref, out_ref):
    x = emb_ref[...]
    m = jnp.max(x, axis=1, keepdims=True)
    s = jnp.sum(jnp.exp(x - m), axis=1, keepdims=True)
    out_ref[...] = m + jnp.log(s)


def _row_lse(emb):
    V = emb.shape[0]
    out = pl.pallas_call(
        _lse_body,
        out_shape=jax.ShapeDtypeStruct((V, 1), jnp.float32),
    )(emb)
    return out.reshape(V)


def _make_sc_gather(N, V, per_w, C):
    nch = per_w // C
    groups = C // LANES
    mesh = plsc.VectorSubcoreMesh(
        core_axis_name="c", subcore_axis_name="s",
        num_cores=NC, num_subcores=NS)

    @functools.partial(
        pl.kernel,
        out_type=(
            jax.ShapeDtypeStruct((N, V), jnp.float32),
            jax.ShapeDtypeStruct((NW, LANES), jnp.float32),
        ),
        mesh=mesh,
        compiler_params=pltpu.CompilerParams(use_tc_tiling_on_sc=False,
                                              needs_layout_passes=False),
        scratch_types=[
            pltpu.VMEM((C,), jnp.int32),
            pltpu.VMEM((C,), jnp.int32),
            pltpu.VMEM((V,), jnp.float32),
            pltpu.VMEM((C, V), jnp.float32),
            pltpu.VMEM((LANES,), jnp.float32),
            pltpu.SemaphoreType.DMA,
        ],
    )
    def sc(emb_hbm, idx_hbm, tgt_hbm, lse_hbm, out_hbm, part_hbm,
           idx_v, tgt_v, lse_v, rows_v, acc_v, sem):
        wid = lax.axis_index("s") * NC + lax.axis_index("c")
        base = wid * per_w
        pltpu.sync_copy(lse_hbm, lse_v)

        def chunk(g, acc):
            off = base + g * C
            pltpu.sync_copy(idx_hbm.at[pl.ds(off, C)], idx_v)
            pltpu.sync_copy(tgt_hbm.at[pl.ds(off, C)], tgt_v)
            pltpu.async_copy(emb_hbm.at[idx_v], rows_v, sem).wait()
            pltpu.sync_copy(rows_v, out_hbm.at[pl.ds(off, C)])
            for sub in range(groups):
                o2 = sub * LANES
                i_vec = lax.iota(jnp.int32, LANES) + o2
                t_vec = tgt_v[pl.ds(o2, LANES)]
                v_vec = idx_v[pl.ds(o2, LANES)]
                val = plsc.load_gather(rows_v, [i_vec, t_vec])
                ls = plsc.load_gather(lse_v, [v_vec])
                acc = acc + (ls - val)
            return acc

        acc = lax.fori_loop(0, nch, chunk, jnp.zeros((LANES,), jnp.float32))
        acc_v[...] = acc
        pltpu.sync_copy(acc_v, part_hbm.at[wid])

    return sc


def kernel(idx, targets, emb):
    B, L = idx.shape
    V = emb.shape[0]
    N = B * L
    per_w = N // NW
    C = 64
    lse = _row_lse(emb)
    idx_f = idx.reshape(N)
    tgt_f = targets.reshape(N)
    sc = _make_sc_gather(N, V, per_w, C)
    logits, part = sc(emb, idx_f, tgt_f, lse)
    loss = jnp.sum(part) / N
    return (logits, loss)


# speedup vs baseline: 1.6911x; 1.0470x over previous
import functools

import jax
import jax.numpy as jnp
from jax import lax
from jax.experimental import pallas as pl
from jax.experimental.pallas import tpu as pltpu
from jax.experimental.pallas import tpu_sc as plsc

NC, NS, LANES = 2, 16, 16
NW = NC * NS


def _lse_body(emb_ref, out_ref):
    x = emb_ref[...]
    m = jnp.max(x, axis=1, keepdims=True)
    s = jnp.sum(jnp.exp(x - m), axis=1, keepdims=True)
    out_ref[...] = m + jnp.log(s)


def _row_lse(emb):
    V = emb.shape[0]
    out = pl.pallas_call(
        _lse_body,
        out_shape=jax.ShapeDtypeStruct((V, 1), jnp.float32),
    )(emb)
    return out.reshape(V)


def _make_sc_gather(N, V, per_w, C):
    nch = per_w // C
    assert nch % 2 == 0 and nch >= 4
    groups = C // LANES
    pairs = (nch - 2) // 2
    mesh = plsc.VectorSubcoreMesh(
        core_axis_name="c", subcore_axis_name="s",
        num_cores=NC, num_subcores=NS)

    @functools.partial(
        pl.kernel,
        out_type=(
            jax.ShapeDtypeStruct((N, V), jnp.float32),
            jax.ShapeDtypeStruct((NW, LANES), jnp.float32),
        ),
        mesh=mesh,
        compiler_params=pltpu.CompilerParams(use_tc_tiling_on_sc=False,
                                              needs_layout_passes=False),
        scratch_types=[
            pltpu.VMEM((per_w,), jnp.int32),
            pltpu.VMEM((per_w,), jnp.int32),
            pltpu.VMEM((V,), jnp.float32),
            pltpu.VMEM((C, V), jnp.float32),
            pltpu.VMEM((C, V), jnp.float32),
            pltpu.VMEM((LANES,), jnp.float32),
            pltpu.SemaphoreType.DMA,
            pltpu.SemaphoreType.DMA,
            pltpu.SemaphoreType.DMA,
        ],
    )
    def sc(emb_hbm, idx_hbm, tgt_hbm, lse_hbm, out_hbm, part_hbm,
           idx_all, tgt_all, lse_v, rows0, rows1, acc_v, gsem, ssem0, ssem1):
        rows = (rows0, rows1)
        ssem = (ssem0, ssem1)
        wid = lax.axis_index("s") * NC + lax.axis_index("c")
        base = wid * per_w
        pltpu.sync_copy(idx_hbm.at[pl.ds(base, per_w)], idx_all)
        pltpu.sync_copy(tgt_hbm.at[pl.ds(base, per_w)], tgt_all)
        pltpu.sync_copy(lse_hbm, lse_v)

        def g_src(loc):
            return emb_hbm.at[idx_all.at[pl.ds(loc, C)]]

        def out_dst(loc):
            return out_hbm.at[pl.ds(base + loc, C)]

        def compute(loc, rbuf, acc):
            for sub in range(groups):
                o2 = loc + sub * LANES
                i_vec = lax.iota(jnp.int32, LANES) + sub * LANES
                t_vec = tgt_all[pl.ds(o2, LANES)]
                v_vec = idx_all[pl.ds(o2, LANES)]
                val = plsc.load_gather(rbuf, [i_vec, t_vec])
                ls = plsc.load_gather(lse_v, [v_vec])
                acc = acc + (ls - val)
            return acc

        pltpu.async_copy(g_src(0), rows0, gsem)
        pltpu.make_async_copy(g_src(0), rows0, gsem).wait()
        pltpu.async_copy(rows0, out_dst(0), ssem0)
        pltpu.async_copy(g_src(C), rows1, gsem)
        acc = compute(0, rows0, jnp.zeros((LANES,), jnp.float32))

        def pair(h, acc):
            for gg in (0, 1):
                s = 2 * h + 1 + gg
                p = 1 - gg
                loc = s * C
                rbuf, obuf = rows[p], rows[1 - p]
                pltpu.make_async_copy(g_src(loc), rbuf, gsem).wait()
                pltpu.async_copy(rbuf, out_dst(loc), ssem[p])
                pltpu.make_async_copy(obuf, out_dst(loc), ssem[1 - p]).wait()
                pltpu.async_copy(g_src(loc + C), obuf, gsem)
                acc = compute(loc, rbuf, acc)
            return acc

        acc = lax.fori_loop(0, pairs, pair, acc)
        loc = (nch - 1) * C
        pltpu.make_async_copy(g_src(loc), rows1, gsem).wait()
        pltpu.async_copy(rows1, out_dst(loc), ssem1)
        pltpu.make_async_copy(rows0, out_dst(loc), ssem0).wait()
        acc = compute(loc, rows1, acc)
        pltpu.make_async_copy(rows1, out_dst(loc), ssem1).wait()

        acc_v[...] = acc
        pltpu.sync_copy(acc_v, part_hbm.at[wid])

    return sc


def kernel(idx, targets, emb):
    B, L = idx.shape
    V = emb.shape[0]
    N = B * L
    per_w = N // NW
    C = 32
    lse = _row_lse(emb)
    idx_f = idx.reshape(N)
    tgt_f = targets.reshape(N)
    sc = _make_sc_gather(N, V, per_w, C)
    logits, part = sc(emb, idx_f, tgt_f, lse)
    loss = jnp.sum(part) / N
    return (logits, loss)
